# baseline (speedup 1.0000x reference)
import functools

import jax
import jax.numpy as jnp
from jax import lax
from jax.experimental import pallas as pl
from jax.experimental.pallas import tpu as pltpu
from jax.experimental.pallas import tpu_sc as plsc

F32 = jnp.float32
I32 = jnp.int32

N_NODES = 10000
HALF = 5000
D = 256
FH = 128
OUT_D = 128
E = 160000

NC = 2
NS = 16
CHUNK = 128
CPS = 80
BLK = 8
NBLK = CPS // BLK
EDGES_PER_TILE = CHUNK * CPS
E_PAD = NS * EDGES_PER_TILE
SLAB = 10240
DUMP = N_NODES
ZPT = SLAB // NS

DCHUNK = 128
DCPS = 80
DEG_SLAB = 5120
DEG_DUMP = DEG_SLAB - 1
DEG_ZPT = DEG_SLAB // NS
DEG_WIN = 8


@functools.lru_cache(maxsize=None)
def _sc_mesh():
    return plsc.VectorSubcoreMesh(core_axis_name="c", subcore_axis_name="s",
                                  num_cores=NC, num_subcores=NS)


def _deg_body(dst_hbm, out_hbm, idx_v, ones_v, zeros_v, deg_sh, sem):
    c = lax.axis_index("c")
    s = lax.axis_index("s")

    def fill(i, _):
        ones_v[i] = jnp.ones((16,), F32)
        return _
    lax.fori_loop(0, DCHUNK, fill, None)

    def fillz(i, _):
        zeros_v[i] = jnp.zeros((16,), F32)
        return _
    lax.fori_loop(0, DEG_ZPT, fillz, None)

    pltpu.sync_copy(zeros_v, deg_sh.at[pl.ds(s * DEG_ZPT, DEG_ZPT)])
    plsc.subcore_barrier()

    pltpu.sync_copy(dst_hbm.at[s], idx_v)

    base = c * HALF

    def body_j(j, _):
        def body_k(k, _):
            v = idx_v[j, pl.ds(k * 16, 16)]
            rel = v - base
            ok = (rel >= 0) & (rel < HALF)
            idx_v[j, pl.ds(k * 16, 16)] = jnp.where(ok, rel, DEG_DUMP)
            return _
        return lax.fori_loop(0, DCHUNK // 16, body_k, _)
    lax.fori_loop(0, DCPS, body_j, None)

    def step(j, _):
        @pl.when(j < DCPS)
        def _fire():
            pltpu.async_copy(ones_v, deg_sh.at[idx_v.at[j]], sem, add=True)

        @pl.when(j >= DEG_WIN)
        def _drain():
            pltpu.make_async_copy(ones_v, deg_sh.at[idx_v.at[0]], sem).wait()
        return _
    lax.fori_loop(0, DCPS + DEG_WIN, step, None)

    plsc.subcore_barrier()
    pltpu.sync_copy(deg_sh.at[pl.ds(s * DEG_ZPT, DEG_ZPT)],
                    out_hbm.at[c, pl.ds(s * DEG_ZPT, DEG_ZPT)])


@functools.lru_cache(maxsize=None)
def _deg_kernel_built():
    return pl.kernel(
        _deg_body,
        out_type=jax.ShapeDtypeStruct((NC, DEG_SLAB, 16), F32),
        mesh=_sc_mesh(),
        scratch_types=[
            pltpu.VMEM((DCPS, DCHUNK), I32),
            pltpu.VMEM((DCHUNK, 16), F32),
            pltpu.VMEM((DEG_ZPT, 16), F32),
            pltpu.VMEM_SHARED((DEG_SLAB, 16), F32),
            pltpu.SemaphoreType.DMA,
        ],
    )


def _edge_body(g_hbm, src_hbm, dst_hbm, out_hbm,
               src_v, dst_v, rows_v, s_sh,
               sg0, sg1, ss0, ss1, sd0, sd1):
    c = lax.axis_index("c")
    s = lax.axis_index("s")
    sgs = (sg0, sg1)
    sss = (ss0, ss1)
    sds = (sd0, sd1)

    def fire_g(j, b):
        pltpu.async_copy(g_hbm.at[c].at[src_v.at[j]], rows_v.at[b], sgs[b])

    def wait_g(b):
        pltpu.make_async_copy(g_hbm.at[c].at[src_v.at[0]], rows_v.at[b],
                              sgs[b]).wait()

    def fire_s(t, bb, b):
        pltpu.async_copy(rows_v.at[b], s_sh.at[dst_v.at[bb].at[t]], sss[b],
                         add=True)

    def wait_s(b):
        pltpu.make_async_copy(g_hbm.at[c].at[src_v.at[0]], rows_v.at[b],
                              sss[b]).wait()

    def fire_d(B, bb):
        pltpu.async_copy(dst_hbm.at[s, pl.ds(B * BLK, BLK)], dst_v.at[bb],
                         sds[bb])

    def wait_d(bb):
        pltpu.make_async_copy(dst_hbm.at[s, pl.ds(0, BLK)], dst_v.at[bb],
                              sds[bb]).wait()

    for b in range(2):
        def fz(i, _, b=b):
            def fk(k, _):
                rows_v[b, i, pl.ds(k * 16, 16)] = jnp.zeros((16,), F32)
                return _
            return lax.fori_loop(0, FH // 16, fk, _)
        lax.fori_loop(0, CHUNK, fz, None)

    def zs(q, _):
        pltpu.sync_copy(rows_v.at[0],
                        s_sh.at[pl.ds(s * ZPT + q * CHUNK, CHUNK)])
        return _
    lax.fori_loop(0, ZPT // CHUNK, zs, None)

    pltpu.sync_copy(src_hbm.at[s], src_v)
    pltpu.sync_copy(dst_hbm.at[s, pl.ds(0, BLK)], dst_v.at[0])
    fire_d(1, 1)
    plsc.subcore_barrier()

    def slot(j, t, bb, first=False):
        b = t % 2
        wait_g(b)
        if t == 0 and not first:
            wait_d(bb)
        fire_s(t, bb, b)
        if not first:
            wait_s(1 - b)
        fire_g(j + 1, 1 - b)

    fire_g(0, 0)
    slot(0, 0, 0, first=True)
    for t in range(1, BLK):
        slot(t, t, 0)
    for t in range(BLK):
        slot(BLK + t, t, 1)
        if t == 1:
            fire_d(2, 0)

    def pair(P, _):
        base = P * 2 * BLK
        for t in range(BLK):
            slot(base + t, t, 0)
            if t == 1:
                fire_d(2 * P + 1, 1)
        for t in range(BLK):
            slot(base + BLK + t, t, 1)
            if t == 1:
                @pl.when(2 * P + 2 < NBLK)
                def _pf():
                    fire_d(2 * P + 2, 0)
        return _
    lax.fori_loop(1, NBLK // 2, pair, None)

    wait_g(CPS % 2)
    wait_s((CPS - 1) % 2)

    plsc.subcore_barrier()
    pltpu.sync_copy(s_sh.at[pl.ds(s * ZPT, ZPT)],
                    out_hbm.at[c, pl.ds(s * ZPT, ZPT)])


@functools.lru_cache(maxsize=None)
def _edge_kernel_built():
    return pl.kernel(
        _edge_body,
        out_type=jax.ShapeDtypeStruct((NC, SLAB, FH), F32),
        mesh=_sc_mesh(),
        scratch_types=[
            pltpu.VMEM((CPS + 1, CHUNK), I32),
            pltpu.VMEM((2, BLK, CHUNK), I32),
            pltpu.VMEM((2, CHUNK, FH), F32),
            pltpu.VMEM_SHARED((SLAB, FH), F32),
            pltpu.SemaphoreType.DMA,
            pltpu.SemaphoreType.DMA,
            pltpu.SemaphoreType.DMA,
            pltpu.SemaphoreType.DMA,
            pltpu.SemaphoreType.DMA,
            pltpu.SemaphoreType.DMA,
        ],
    )


def _proj_body(x_ref, ws_ref, bs_ref, w1_ref, deg_ref, o_ref):
    x = x_ref[...]
    h = jnp.dot(x, ws_ref[0], preferred_element_type=F32) + bs_ref[0]
    h = jnp.maximum(h, 0.0)
    dinv = lax.rsqrt(deg_ref[...] + 1.0)
    g = jnp.dot(h, w1_ref[...], preferred_element_type=F32) * dinv
    o_ref[0] = g[:, :FH]
    o_ref[1] = g[:, FH:]


def _mid_body(s_ref, g_ref, deg_ref, b_ref, w_ref, o_ref):
    dinv = lax.rsqrt(deg_ref[...] + 1.0)
    s_mat = jnp.concatenate([s_ref[0], s_ref[1]], axis=1)
    g_mat = jnp.concatenate([g_ref[0], g_ref[1]], axis=1)
    x = jnp.maximum(dinv * (s_mat + g_mat) + b_ref[...], 0.0)
    g = jnp.dot(x, w_ref[...], preferred_element_type=F32) * dinv
    o_ref[0] = g[:, :FH]
    o_ref[1] = g[:, FH:]


def _fin_body(s_ref, g_ref, deg_ref, b_ref, w_ref, bo_ref, o_ref):
    dinv = lax.rsqrt(deg_ref[...] + 1.0)
    s_mat = jnp.concatenate([s_ref[0], s_ref[1]], axis=1)
    g_mat = jnp.concatenate([g_ref[0], g_ref[1]], axis=1)
    x = jnp.maximum(dinv * (s_mat + g_mat) + b_ref[...], 0.0)
    o_ref[...] = jnp.dot(x, w_ref[...], preferred_element_type=F32) + bo_ref[...]


_RB = 1000


def _proj_call(x_cat, w_s, b_s, w1, deg):
    grid = N_NODES // _RB
    return pl.pallas_call(
        _proj_body,
        grid=(grid,),
        in_specs=[
            pl.BlockSpec((_RB, D), lambda i: (i, 0)),
            pl.BlockSpec((1, D, D), lambda i: (i // (grid // 2), 0, 0)),
            pl.BlockSpec((1, 1, D), lambda i: (i // (grid // 2), 0, 0)),
            pl.BlockSpec((D, D), lambda i: (0, 0)),
            pl.BlockSpec((_RB, 1), lambda i: (i, 0)),
        ],
        out_specs=pl.BlockSpec((NC, _RB, FH), lambda i: (0, i, 0)),
        out_shape=jax.ShapeDtypeStruct((NC, N_NODES, FH), F32),
    )(x_cat, w_s, b_s, w1, deg)


def _mid_call(slabs, g, deg, b, w):
    grid = N_NODES // _RB
    return pl.pallas_call(
        _mid_body,
        grid=(grid,),
        in_specs=[
            pl.BlockSpec((NC, _RB, FH), lambda i: (0, i, 0)),
            pl.BlockSpec((NC, _RB, FH), lambda i: (0, i, 0)),
            pl.BlockSpec((_RB, 1), lambda i: (i, 0)),
            pl.BlockSpec((1, D), lambda i: (0, 0)),
            pl.BlockSpec((D, D), lambda i: (0, 0)),
        ],
        out_specs=pl.BlockSpec((NC, _RB, FH), lambda i: (0, i, 0)),
        out_shape=jax.ShapeDtypeStruct((NC, N_NODES, FH), F32),
    )(slabs, g, deg, b, w)


def _fin_call(slabs, g, deg, b, w, bo):
    grid = HALF // _RB
    return pl.pallas_call(
        _fin_body,
        grid=(grid,),
        in_specs=[
            pl.BlockSpec((NC, _RB, FH), lambda i: (0, i, 0)),
            pl.BlockSpec((NC, _RB, FH), lambda i: (0, i, 0)),
            pl.BlockSpec((_RB, 1), lambda i: (i, 0)),
            pl.BlockSpec((1, D), lambda i: (0, 0)),
            pl.BlockSpec((D, OUT_D), lambda i: (0, 0)),
            pl.BlockSpec((1, OUT_D), lambda i: (0, 0)),
        ],
        out_specs=pl.BlockSpec((_RB, OUT_D), lambda i: (i, 0)),
        out_shape=jax.ShapeDtypeStruct((HALF, OUT_D), F32),
    )(slabs, g, deg, b, w, bo)


def kernel(x_u, x_p, edge_index, W_u, b_u, W_p, b_p, W1, b1, W2, b2, W_out, b_out):
    src = edge_index[0]
    dst = edge_index[1]
    pad = E_PAD - E
    src_p = jnp.concatenate([src, jnp.zeros((pad,), I32)])
    dst_p = jnp.concatenate([dst, jnp.full((pad,), jnp.int32(DUMP))])
    src_r = jnp.pad(src_p.reshape(NS, CPS, CHUNK), ((0, 0), (0, 1), (0, 0)))
    dst_r = dst_p.reshape(NS, CPS, CHUNK)
    dst_d = dst_p.reshape(NS, DCPS, DCHUNK)

    deg_slabs = _deg_kernel_built()(dst_d)
    deg = jnp.concatenate([deg_slabs[0, :HALF, 0], deg_slabs[1, :HALF, 0]]
                          )[:, None]

    x_cat = jnp.concatenate([x_u, x_p], axis=0)
    w_s = jnp.stack([W_u, W_p])
    b_s = jnp.stack([b_u, b_p])[:, None, :]

    g1 = _proj_call(x_cat, w_s, b_s, W1, deg)
    s1 = _edge_kernel_built()(g1, src_r, dst_r)
    g2 = _mid_call(s1, g1, deg, b1[None, :], W2)
    s2 = _edge_kernel_built()(g2, src_r, dst_r)
    return _fin_call(s2, g2, deg, b2[None, :], W_out, b_out[None, :])


# speedup vs baseline: 1.0016x; 1.0016x over previous
import functools

import jax
import jax.numpy as jnp
from jax import lax
from jax.experimental import pallas as pl
from jax.experimental.pallas import tpu as pltpu
from jax.experimental.pallas import tpu_sc as plsc

F32 = jnp.float32
I32 = jnp.int32

N_NODES = 10000
HALF = 5000
D = 256
FH = 128
OUT_D = 128
E = 160000

NC = 2
NS = 16
CHUNK = 128
CPS = 80
BLK = 8
NBLK = CPS // BLK
EDGES_PER_TILE = CHUNK * CPS
E_PAD = NS * EDGES_PER_TILE
SLAB = 10240
DUMP = N_NODES
ZPT = SLAB // NS

DCHUNK = 128
DCPS = 80
DEG_SLAB = 5120
DEG_DUMP = DEG_SLAB - 1
DEG_ZPT = DEG_SLAB // NS
DEG_WIN = 8


@functools.lru_cache(maxsize=None)
def _sc_mesh():
    return plsc.VectorSubcoreMesh(core_axis_name="c", subcore_axis_name="s",
                                  num_cores=NC, num_subcores=NS)


def _deg_body(dst_hbm, out_hbm, idx_v, ones_v, zeros_v, deg_sh, sem):
    c = lax.axis_index("c")
    s = lax.axis_index("s")

    def fill(i, _):
        ones_v[i] = jnp.ones((16,), F32)
        return _
    lax.fori_loop(0, DCHUNK, fill, None)

    def fillz(i, _):
        zeros_v[i] = jnp.zeros((16,), F32)
        return _
    lax.fori_loop(0, DEG_ZPT, fillz, None)

    pltpu.sync_copy(zeros_v, deg_sh.at[pl.ds(s * DEG_ZPT, DEG_ZPT)])
    plsc.subcore_barrier()

    pltpu.sync_copy(dst_hbm.at[s], idx_v)

    base = c * HALF

    def body_j(j, _):
        def body_k(k, _):
            v = idx_v[j, pl.ds(k * 16, 16)]
            rel = v - base
            ok = (rel >= 0) & (rel < HALF)
            idx_v[j, pl.ds(k * 16, 16)] = jnp.where(ok, rel, DEG_DUMP)
            return _
        return lax.fori_loop(0, DCHUNK // 16, body_k, _)
    lax.fori_loop(0, DCPS, body_j, None)

    def step(j, _):
        pltpu.async_copy(ones_v, deg_sh.at[idx_v.at[j]], sem, add=True)
        return _
    lax.fori_loop(0, DCPS, step, None)

    drain = pltpu.make_async_copy(ones_v, deg_sh.at[idx_v.at[0]], sem)

    def dstep(j, _):
        drain.wait()
        return _
    lax.fori_loop(0, DCPS, dstep, None)

    plsc.subcore_barrier()
    pltpu.sync_copy(deg_sh.at[pl.ds(s * DEG_ZPT, DEG_ZPT)],
                    out_hbm.at[c, pl.ds(s * DEG_ZPT, DEG_ZPT)])


@functools.lru_cache(maxsize=None)
def _deg_kernel_built():
    return pl.kernel(
        _deg_body,
        out_type=jax.ShapeDtypeStruct((NC, DEG_SLAB, 16), F32),
        mesh=_sc_mesh(),
        scratch_types=[
            pltpu.VMEM((DCPS, DCHUNK), I32),
            pltpu.VMEM((DCHUNK, 16), F32),
            pltpu.VMEM((DEG_ZPT, 16), F32),
            pltpu.VMEM_SHARED((DEG_SLAB, 16), F32),
            pltpu.SemaphoreType.DMA,
        ],
    )


def _edge_body(g_hbm, src_hbm, dst_hbm, out_hbm,
               src_v, dst_v, rows_v, s_sh,
               sg0, sg1, ss0, ss1, sd0, sd1):
    c = lax.axis_index("c")
    s = lax.axis_index("s")
    sgs = (sg0, sg1)
    sss = (ss0, ss1)
    sds = (sd0, sd1)

    wgd = [pltpu.make_async_copy(g_hbm.at[c].at[src_v.at[0]], rows_v.at[b],
                                 sgs[b]) for b in range(2)]
    wsd = [pltpu.make_async_copy(g_hbm.at[c].at[src_v.at[0]], rows_v.at[b],
                                 sss[b]) for b in range(2)]
    wdd = [pltpu.make_async_copy(dst_hbm.at[s, pl.ds(0, BLK)], dst_v.at[bb],
                                 sds[bb]) for bb in range(2)]

    def fire_g(j, b):
        pltpu.async_copy(g_hbm.at[c].at[src_v.at[j]], rows_v.at[b], sgs[b])

    def fire_s(t, bb, b):
        pltpu.async_copy(rows_v.at[b], s_sh.at[dst_v.at[bb].at[t]], sss[b],
                         add=True)

    def fire_d(B, bb):
        pltpu.async_copy(dst_hbm.at[s, pl.ds(B * BLK, BLK)], dst_v.at[bb],
                         sds[bb])

    for b in range(2):
        def fz(i, _, b=b):
            def fk(k, _):
                rows_v[b, i, pl.ds(k * 16, 16)] = jnp.zeros((16,), F32)
                return _
            return lax.fori_loop(0, FH // 16, fk, _)
        lax.fori_loop(0, CHUNK, fz, None)

    def zs(q, _):
        pltpu.sync_copy(rows_v.at[0],
                        s_sh.at[pl.ds(s * ZPT + q * CHUNK, CHUNK)])
        return _
    lax.fori_loop(0, ZPT // CHUNK, zs, None)

    pltpu.sync_copy(src_hbm.at[s], src_v)
    pltpu.sync_copy(dst_hbm.at[s, pl.ds(0, BLK)], dst_v.at[0])
    fire_d(1, 1)
    plsc.subcore_barrier()

    def slot(j, t, bb, first=False):
        b = t % 2
        wgd[b].wait()
        if t == 0 and not first:
            wdd[bb].wait()
        fire_s(t, bb, b)
        if not first:
            wsd[1 - b].wait()
        fire_g(j + 1, 1 - b)

    fire_g(0, 0)
    slot(0, 0, 0, first=True)
    for t in range(1, BLK):
        slot(t, t, 0)
    for t in range(BLK):
        slot(BLK + t, t, 1)
        if t == 1:
            fire_d(2, 0)

    def pair(P, _):
        base = P * 2 * BLK
        for t in range(BLK):
            slot(base + t, t, 0)
            if t == 1:
                fire_d(2 * P + 1, 1)
        for t in range(BLK):
            slot(base + BLK + t, t, 1)
            if t == 1:
                @pl.when(2 * P + 2 < NBLK)
                def _pf():
                    fire_d(2 * P + 2, 0)
        return _
    lax.fori_loop(1, NBLK // 2, pair, None)

    wgd[CPS % 2].wait()
    wsd[(CPS - 1) % 2].wait()

    plsc.subcore_barrier()
    pltpu.sync_copy(s_sh.at[pl.ds(s * ZPT, ZPT)],
                    out_hbm.at[c, pl.ds(s * ZPT, ZPT)])


@functools.lru_cache(maxsize=None)
def _edge_kernel_built():
    return pl.kernel(
        _edge_body,
        out_type=jax.ShapeDtypeStruct((NC, SLAB, FH), F32),
        mesh=_sc_mesh(),
        scratch_types=[
            pltpu.VMEM((CPS + 1, CHUNK), I32),
            pltpu.VMEM((2, BLK, CHUNK), I32),
            pltpu.VMEM((2, CHUNK, FH), F32),
            pltpu.VMEM_SHARED((SLAB, FH), F32),
            pltpu.SemaphoreType.DMA,
            pltpu.SemaphoreType.DMA,
            pltpu.SemaphoreType.DMA,
            pltpu.SemaphoreType.DMA,
            pltpu.SemaphoreType.DMA,
            pltpu.SemaphoreType.DMA,
        ],
    )


def _proj_body(x_ref, ws_ref, bs_ref, w1_ref, deg_ref, o_ref):
    x = x_ref[...]
    h = jnp.dot(x, ws_ref[0], preferred_element_type=F32) + bs_ref[0]
    h = jnp.maximum(h, 0.0)
    dinv = lax.rsqrt(deg_ref[...] + 1.0)
    g = jnp.dot(h, w1_ref[...], preferred_element_type=F32) * dinv
    o_ref[0] = g[:, :FH]
    o_ref[1] = g[:, FH:]


def _mid_body(s_ref, g_ref, deg_ref, b_ref, w_ref, o_ref):
    dinv = lax.rsqrt(deg_ref[...] + 1.0)
    s_mat = jnp.concatenate([s_ref[0], s_ref[1]], axis=1)
    g_mat = jnp.concatenate([g_ref[0], g_ref[1]], axis=1)
    x = jnp.maximum(dinv * (s_mat + g_mat) + b_ref[...], 0.0)
    g = jnp.dot(x, w_ref[...], preferred_element_type=F32) * dinv
    o_ref[0] = g[:, :FH]
    o_ref[1] = g[:, FH:]


def _fin_body(s_ref, g_ref, deg_ref, b_ref, w_ref, bo_ref, o_ref):
    dinv = lax.rsqrt(deg_ref[...] + 1.0)
    s_mat = jnp.concatenate([s_ref[0], s_ref[1]], axis=1)
    g_mat = jnp.concatenate([g_ref[0], g_ref[1]], axis=1)
    x = jnp.maximum(dinv * (s_mat + g_mat) + b_ref[...], 0.0)
    o_ref[...] = jnp.dot(x, w_ref[...], preferred_element_type=F32) + bo_ref[...]


_RB = 1000


def _proj_call(x_cat, w_s, b_s, w1, deg):
    grid = N_NODES // _RB
    return pl.pallas_call(
        _proj_body,
        grid=(grid,),
        in_specs=[
            pl.BlockSpec((_RB, D), lambda i: (i, 0)),
            pl.BlockSpec((1, D, D), lambda i: (i // (grid // 2), 0, 0)),
            pl.BlockSpec((1, 1, D), lambda i: (i // (grid // 2), 0, 0)),
            pl.BlockSpec((D, D), lambda i: (0, 0)),
            pl.BlockSpec((_RB, 1), lambda i: (i, 0)),
        ],
        out_specs=pl.BlockSpec((NC, _RB, FH), lambda i: (0, i, 0)),
        out_shape=jax.ShapeDtypeStruct((NC, N_NODES, FH), F32),
    )(x_cat, w_s, b_s, w1, deg)


def _mid_call(slabs, g, deg, b, w):
    grid = N_NODES // _RB
    return pl.pallas_call(
        _mid_body,
        grid=(grid,),
        in_specs=[
            pl.BlockSpec((NC, _RB, FH), lambda i: (0, i, 0)),
            pl.BlockSpec((NC, _RB, FH), lambda i: (0, i, 0)),
            pl.BlockSpec((_RB, 1), lambda i: (i, 0)),
            pl.BlockSpec((1, D), lambda i: (0, 0)),
            pl.BlockSpec((D, D), lambda i: (0, 0)),
        ],
        out_specs=pl.BlockSpec((NC, _RB, FH), lambda i: (0, i, 0)),
        out_shape=jax.ShapeDtypeStruct((NC, N_NODES, FH), F32),
    )(slabs, g, deg, b, w)


def _fin_call(slabs, g, deg, b, w, bo):
    grid = HALF // _RB
    return pl.pallas_call(
        _fin_body,
        grid=(grid,),
        in_specs=[
            pl.BlockSpec((NC, _RB, FH), lambda i: (0, i, 0)),
            pl.BlockSpec((NC, _RB, FH), lambda i: (0, i, 0)),
            pl.BlockSpec((_RB, 1), lambda i: (i, 0)),
            pl.BlockSpec((1, D), lambda i: (0, 0)),
            pl.BlockSpec((D, OUT_D), lambda i: (0, 0)),
            pl.BlockSpec((1, OUT_D), lambda i: (0, 0)),
        ],
        out_specs=pl.BlockSpec((_RB, OUT_D), lambda i: (i, 0)),
        out_shape=jax.ShapeDtypeStruct((HALF, OUT_D), F32),
    )(slabs, g, deg, b, w, bo)


def kernel(x_u, x_p, edge_index, W_u, b_u, W_p, b_p, W1, b1, W2, b2, W_out, b_out):
    src = edge_index[0]
    dst = edge_index[1]
    pad = E_PAD - E
    src_p = jnp.concatenate([src, jnp.zeros((pad,), I32)])
    dst_p = jnp.concatenate([dst, jnp.full((pad,), jnp.int32(DUMP))])
    src_r = jnp.pad(src_p.reshape(NS, CPS, CHUNK), ((0, 0), (0, 1), (0, 0)))
    dst_r = dst_p.reshape(NS, CPS, CHUNK)
    dst_d = dst_p.reshape(NS, DCPS, DCHUNK)

    deg_slabs = _deg_kernel_built()(dst_d)
    deg = jnp.concatenate([deg_slabs[0, :HALF, 0], deg_slabs[1, :HALF, 0]]
                          )[:, None]

    x_cat = jnp.concatenate([x_u, x_p], axis=0)
    w_s = jnp.stack([W_u, W_p])
    b_s = jnp.stack([b_u, b_p])[:, None, :]

    g1 = _proj_call(x_cat, w_s, b_s, W1, deg)
    s1 = _edge_kernel_built()(g1, src_r, dst_r)
    g2 = _mid_call(s1, g1, deg, b1[None, :], W2)
    s2 = _edge_kernel_built()(g2, src_r, dst_r)
    return _fin_call(s2, g2, deg, b2[None, :], W_out, b_out[None, :])


# speedup vs baseline: 1.1594x; 1.1575x over previous
import functools

import jax
import jax.numpy as jnp
from jax import lax
from jax.experimental import pallas as pl
from jax.experimental.pallas import tpu as pltpu
from jax.experimental.pallas import tpu_sc as plsc

F32 = jnp.float32
I32 = jnp.int32

N_NODES = 10000
HALF = 5000
D = 256
FH = 128
OUT_D = 128
E = 160000

NC = 2
NS = 16
CHUNK = 128
CPS = 80
BLK = 8
NBLK = CPS // BLK
EDGES_PER_TILE = CHUNK * CPS
E_PAD = NS * EDGES_PER_TILE
SLAB = 10240
DUMP = N_NODES
ZPT = SLAB // NS

DCHUNK = 128
DCPS = 80
DEG_SLAB = 5120
DEG_DUMP = DEG_SLAB - 1
DEG_ZPT = DEG_SLAB // NS


@functools.lru_cache(maxsize=None)
def _sc_mesh():
    return plsc.VectorSubcoreMesh(core_axis_name="c", subcore_axis_name="s",
                                  num_cores=NC, num_subcores=NS)


def _deg_body(dst_hbm, out_hbm, idx_v, ones_v, zeros_v, deg_sh, sem):
    c = lax.axis_index("c")
    s = lax.axis_index("s")

    def fill(i, _):
        ones_v[i] = jnp.ones((16,), F32)
        return _
    lax.fori_loop(0, DCHUNK, fill, None)

    def fillz(i, _):
        zeros_v[i] = jnp.zeros((16,), F32)
        return _
    lax.fori_loop(0, DEG_ZPT, fillz, None)

    pltpu.sync_copy(zeros_v, deg_sh.at[pl.ds(s * DEG_ZPT, DEG_ZPT)])
    plsc.subcore_barrier()

    pltpu.sync_copy(dst_hbm.at[s], idx_v)

    base = c * HALF

    def body_j(j, _):
        def body_k(k, _):
            v = idx_v[j, pl.ds(k * 16, 16)]
            rel = v - base
            ok = (rel >= 0) & (rel < HALF)
            idx_v[j, pl.ds(k * 16, 16)] = jnp.where(ok, rel, DEG_DUMP)
            return _
        return lax.fori_loop(0, DCHUNK // 16, body_k, _)
    lax.fori_loop(0, DCPS, body_j, None)

    def step(j, _):
        pltpu.async_copy(ones_v, deg_sh.at[idx_v.at[j]], sem, add=True)
        return _
    lax.fori_loop(0, DCPS, step, None)

    def dstep(j, _):
        pltpu.make_async_copy(ones_v, deg_sh.at[idx_v.at[0]], sem).wait()
        return _
    lax.fori_loop(0, DCPS, dstep, None)

    plsc.subcore_barrier()
    pltpu.sync_copy(deg_sh.at[pl.ds(s * DEG_ZPT, DEG_ZPT)],
                    out_hbm.at[c, pl.ds(s * DEG_ZPT, DEG_ZPT)])

@functools.lru_cache(maxsize=None)
def _deg_kernel_built():
    return pl.kernel(
        _deg_body,
        out_type=jax.ShapeDtypeStruct((NC, DEG_SLAB, 16), F32),
        mesh=_sc_mesh(),
        scratch_types=[
            pltpu.VMEM((DCPS, DCHUNK), I32),
            pltpu.VMEM((DCHUNK, 16), F32),
            pltpu.VMEM((DEG_ZPT, 16), F32),
            pltpu.VMEM_SHARED((DEG_SLAB, 16), F32),
            pltpu.SemaphoreType.DMA,
        ],
    )


def _edge_body(g_hbm, src_hbm, dst_hbm, out_hbm,
               src_v, dst_v, rows_v, s_sh, sg):
    c = lax.axis_index("c")
    s = lax.axis_index("s")

    def fz(i, _):
        def fk(k, _):
            rows_v[i, pl.ds(k * 16, 16)] = jnp.zeros((16,), F32)
            return _
        return lax.fori_loop(0, FH // 16, fk, _)
    lax.fori_loop(0, CHUNK, fz, None)

    def zs(q, _):
        pltpu.sync_copy(rows_v,
                        s_sh.at[pl.ds(s * ZPT + q * CHUNK, CHUNK)])
        return _
    lax.fori_loop(0, ZPT // CHUNK, zs, None)

    pltpu.sync_copy(src_hbm.at[s], src_v)
    pltpu.sync_copy(dst_hbm.at[s], dst_v)
    plsc.subcore_barrier()

    def step(j, _):
        pltpu.async_copy(g_hbm.at[c].at[src_v.at[j]], rows_v, sg).wait()
        pltpu.sync_copy(rows_v, s_sh.at[dst_v.at[j]], add=True)
        return _
    lax.fori_loop(0, CPS, step, None)

    plsc.subcore_barrier()
    pltpu.sync_copy(s_sh.at[pl.ds(s * ZPT, ZPT)],
                    out_hbm.at[c, pl.ds(s * ZPT, ZPT)])


@functools.lru_cache(maxsize=None)
def _edge_kernel_built():
    return pl.kernel(
        _edge_body,
        out_type=jax.ShapeDtypeStruct((NC, SLAB, FH), F32),
        mesh=_sc_mesh(),
        scratch_types=[
            pltpu.VMEM((CPS, CHUNK), I32),
            pltpu.VMEM((CPS, CHUNK), I32),
            pltpu.VMEM((CHUNK, FH), F32),
            pltpu.VMEM_SHARED((SLAB, FH), F32),
            pltpu.SemaphoreType.DMA,
        ],
    )


def _dinv_body(deg_ref, o_ref):
    o_ref[...] = lax.rsqrt(deg_ref[0][:, :1] + 1.0)


def _dinv_call(deg):
    grid = N_NODES // _RB
    return pl.pallas_call(
        _dinv_body,
        grid=(grid,),
        in_specs=[
            pl.BlockSpec((1, _RB, 16),
                         lambda i: (i // (grid // 2), i % (grid // 2), 0)),
        ],
        out_specs=pl.BlockSpec((_RB, 1), lambda i: (i, 0)),
        out_shape=jax.ShapeDtypeStruct((N_NODES, 1), F32),
    )(deg)


def _proj_body(x_ref, ws_ref, bs_ref, w1_ref, dinv_ref, o_ref):
    x = x_ref[...]
    h = jnp.dot(x, ws_ref[0], preferred_element_type=F32) + bs_ref[0]
    h = jnp.maximum(h, 0.0)
    dinv = dinv_ref[...]
    g = jnp.dot(h, w1_ref[...], preferred_element_type=F32) * dinv
    o_ref[0] = g[:, :FH]
    o_ref[1] = g[:, FH:]


def _mid_body(s_ref, g_ref, dinv_ref, b_ref, w_ref, o_ref):
    dinv = dinv_ref[...]
    s_mat = jnp.concatenate([s_ref[0], s_ref[1]], axis=1)
    g_mat = jnp.concatenate([g_ref[0], g_ref[1]], axis=1)
    x = jnp.maximum(dinv * (s_mat + g_mat) + b_ref[...], 0.0)
    g = jnp.dot(x, w_ref[...], preferred_element_type=F32) * dinv
    o_ref[0] = g[:, :FH]
    o_ref[1] = g[:, FH:]


def _fin_body(s_ref, g_ref, dinv_ref, b_ref, w_ref, bo_ref, o_ref):
    dinv = dinv_ref[...]
    s_mat = jnp.concatenate([s_ref[0], s_ref[1]], axis=1)
    g_mat = jnp.concatenate([g_ref[0], g_ref[1]], axis=1)
    x = jnp.maximum(dinv * (s_mat + g_mat) + b_ref[...], 0.0)
    o_ref[...] = jnp.dot(x, w_ref[...], preferred_element_type=F32) + bo_ref[...]


_RB = 1000


def _proj_call(x_cat, w_s, b_s, w1, deg):
    grid = N_NODES // _RB
    return pl.pallas_call(
        _proj_body,
        grid=(grid,),
        in_specs=[
            pl.BlockSpec((_RB, D), lambda i: (i, 0)),
            pl.BlockSpec((1, D, D), lambda i: (i // (grid // 2), 0, 0)),
            pl.BlockSpec((1, 1, D), lambda i: (i // (grid // 2), 0, 0)),
            pl.BlockSpec((D, D), lambda i: (0, 0)),
            pl.BlockSpec((_RB, 1), lambda i: (i, 0)),
        ],
        out_specs=pl.BlockSpec((NC, _RB, FH), lambda i: (0, i, 0)),
        out_shape=jax.ShapeDtypeStruct((NC, N_NODES, FH), F32),
    )(x_cat, w_s, b_s, w1, deg)


def _mid_call(slabs, g, deg, b, w):
    grid = N_NODES // _RB
    return pl.pallas_call(
        _mid_body,
        grid=(grid,),
        in_specs=[
            pl.BlockSpec((NC, _RB, FH), lambda i: (0, i, 0)),
            pl.BlockSpec((NC, _RB, FH), lambda i: (0, i, 0)),
            pl.BlockSpec((_RB, 1), lambda i: (i, 0)),
            pl.BlockSpec((1, D), lambda i: (0, 0)),
            pl.BlockSpec((D, D), lambda i: (0, 0)),
        ],
        out_specs=pl.BlockSpec((NC, _RB, FH), lambda i: (0, i, 0)),
        out_shape=jax.ShapeDtypeStruct((NC, N_NODES, FH), F32),
    )(slabs, g, deg, b, w)


def _fin_call(slabs, g, deg, b, w, bo):
    grid = HALF // _RB
    return pl.pallas_call(
        _fin_body,
        grid=(grid,),
        in_specs=[
            pl.BlockSpec((NC, _RB, FH), lambda i: (0, i, 0)),
            pl.BlockSpec((NC, _RB, FH), lambda i: (0, i, 0)),
            pl.BlockSpec((_RB, 1), lambda i: (i, 0)),
            pl.BlockSpec((1, D), lambda i: (0, 0)),
            pl.BlockSpec((D, OUT_D), lambda i: (0, 0)),
            pl.BlockSpec((1, OUT_D), lambda i: (0, 0)),
        ],
        out_specs=pl.BlockSpec((_RB, OUT_D), lambda i: (i, 0)),
        out_shape=jax.ShapeDtypeStruct((HALF, OUT_D), F32),
    )(slabs, g, deg, b, w, bo)


def kernel(x_u, x_p, edge_index, W_u, b_u, W_p, b_p, W1, b1, W2, b2, W_out, b_out):
    src = edge_index[0]
    dst = edge_index[1]
    pad = E_PAD - E
    src_p = jnp.concatenate([src, jnp.zeros((pad,), I32)])
    dst_p = jnp.concatenate([dst, jnp.full((pad,), jnp.int32(DUMP))])
    src_r = src_p.reshape(NS, CPS, CHUNK)
    dst_r = dst_p.reshape(NS, CPS, CHUNK)
    dst_d = dst_p.reshape(NS, DCPS, DCHUNK)

    deg = _dinv_call(_deg_kernel_built()(dst_d))

    x_cat = jnp.concatenate([x_u, x_p], axis=0)
    w_s = jnp.stack([W_u, W_p])
    b_s = jnp.stack([b_u, b_p])[:, None, :]

    g1 = _proj_call(x_cat, w_s, b_s, W1, deg)
    s1 = _edge_kernel_built()(g1, src_r, dst_r)
    g2 = _mid_call(s1, g1, deg, b1[None, :], W2)
    s2 = _edge_kernel_built()(g2, src_r, dst_r)
    return _fin_call(s2, g2, deg, b2[None, :], W_out, b_out[None, :])
